# baseline (speedup 1.0000x reference)
import numpy as np

import jax
import jax.numpy as jnp
from jax.experimental import pallas as pl
from jax.experimental.pallas import tpu as pltpu

_TEMP = 0.7
_ROWS = 128
_COLS = 100000
_BLOCK = 2048
_NBLK = (_COLS + _BLOCK - 1) // _BLOCK
_PAD_COLS = _NBLK * _BLOCK
_TINY = float(jnp.finfo(jnp.float32).tiny)


def _threefry_bits_table():
    ks1 = np.uint32(42)
    ks2 = np.uint32(0x1BD11BDA) ^ ks1
    rot = ((13, 15, 26, 6), (17, 29, 16, 24))
    inj = ((ks1, ks2), (ks2, np.uint32(0)), (np.uint32(0), ks1),
           (ks1, ks2), (ks2, np.uint32(0)))
    with np.errstate(over="ignore"):
        ctr = np.arange(_ROWS * _COLS, dtype=np.uint32)
        x0 = np.zeros_like(ctr)
        x1 = ctr + ks1
        for i in range(5):
            for r in rot[i % 2]:
                x0 += x1
                x1 = (x1 << np.uint32(r)) | (x1 >> np.uint32(32 - r))
                x1 ^= x0
            x0 += inj[i][0]
            x1 += inj[i][1] + np.uint32(i + 1)
        bits = (x0 ^ x1).reshape(_ROWS, _COLS)
    if _PAD_COLS != _COLS:
        bits = np.pad(bits, ((0, 0), (0, _PAD_COLS - _COLS)))
    return np.ascontiguousarray(
        bits.reshape(_ROWS, _NBLK, _BLOCK).transpose(1, 0, 2))


_BITS = _threefry_bits_table()


def _sample_kernel(x_ref, bits_ref, out_ref, max_ref, arg_ref):
    b = pl.program_id(0)

    bits = bits_ref[0]
    fb = (bits >> jnp.uint32(9)) | jnp.uint32(0x3F800000)
    u = jax.lax.bitcast_convert_type(fb, jnp.float32) - jnp.float32(1.0)
    u = jnp.maximum(jnp.float32(_TINY), u + jnp.float32(_TINY))
    g = -jnp.log(-jnp.log(u))

    s = x_ref[...] * jnp.float32(_TEMP) + g

    col = jax.lax.broadcasted_iota(jnp.int32, (_ROWS, _BLOCK), 1) + b * _BLOCK
    s = jnp.where(col < _COLS, s, jnp.float32(float("-inf")))

    m = jnp.max(s, axis=1, keepdims=True)
    a = jnp.min(jnp.where(s == m, col, jnp.int32(2**31 - 1)),
                axis=1, keepdims=True)

    @pl.when(b == 0)
    def _():
        max_ref[...] = m
        arg_ref[...] = a

    @pl.when(b > 0)
    def _():
        upd = m > max_ref[...]
        arg_ref[...] = jnp.where(upd, a, arg_ref[...])
        max_ref[...] = jnp.maximum(m, max_ref[...])

    @pl.when(b == _NBLK - 1)
    def _():
        out_ref[...] = arg_ref[...]


@jax.jit
def kernel(outputs):
    actions = pl.pallas_call(
        _sample_kernel,
        grid=(_NBLK,),
        in_specs=[
            pl.BlockSpec((_ROWS, _BLOCK), lambda b: (0, b)),
            pl.BlockSpec((1, _ROWS, _BLOCK), lambda b: (b, 0, 0)),
        ],
        out_specs=pl.BlockSpec((_ROWS, 1), lambda b: (0, 0)),
        out_shape=jax.ShapeDtypeStruct((_ROWS, 1), jnp.int32),
        scratch_shapes=[
            pltpu.VMEM((_ROWS, 1), jnp.float32),
            pltpu.VMEM((_ROWS, 1), jnp.int32),
        ],
    )(outputs, _BITS)
    return actions


# speedup vs baseline: 1.1332x; 1.1332x over previous
import numpy as np

import jax
import jax.numpy as jnp
from jax.experimental import pallas as pl
from jax.experimental.pallas import tpu as pltpu

_TEMP = 0.7
_ROWS = 128
_COLS = 100000
_BLOCK = 4096
_NBLK = (_COLS + _BLOCK - 1) // _BLOCK
_PAD_COLS = _NBLK * _BLOCK
_TINY = float(jnp.finfo(jnp.float32).tiny)


def _threefry_bits_table():
    ks1 = np.uint32(42)
    ks2 = np.uint32(0x1BD11BDA) ^ ks1
    rot = ((13, 15, 26, 6), (17, 29, 16, 24))
    inj = ((ks1, ks2), (ks2, np.uint32(0)), (np.uint32(0), ks1),
           (ks1, ks2), (ks2, np.uint32(0)))
    with np.errstate(over="ignore"):
        ctr = np.arange(_ROWS * _COLS, dtype=np.uint32)
        x0 = np.zeros_like(ctr)
        x1 = ctr + ks1
        for i in range(5):
            for r in rot[i % 2]:
                x0 += x1
                x1 = (x1 << np.uint32(r)) | (x1 >> np.uint32(32 - r))
                x1 ^= x0
            x0 += inj[i][0]
            x1 += inj[i][1] + np.uint32(i + 1)
        bits = (x0 ^ x1).reshape(_ROWS, _COLS)
    if _PAD_COLS != _COLS:
        bits = np.pad(bits, ((0, 0), (0, _PAD_COLS - _COLS)))
    return np.ascontiguousarray(
        bits.reshape(_ROWS, _NBLK, _BLOCK).transpose(1, 0, 2))


_BITS = _threefry_bits_table()


def _sample_kernel(x_ref, bits_ref, out_ref, max_ref, arg_ref):
    b = pl.program_id(0)

    bits = bits_ref[0]
    fb = (bits >> jnp.uint32(9)) | jnp.uint32(0x3F800000)
    u = jax.lax.bitcast_convert_type(fb, jnp.float32) - jnp.float32(1.0)
    u = jnp.maximum(jnp.float32(_TINY), u + jnp.float32(_TINY))
    g = -jnp.log(-jnp.log(u))

    s = x_ref[...] * jnp.float32(_TEMP) + g

    col = jax.lax.broadcasted_iota(jnp.int32, (_ROWS, _BLOCK), 1) + b * _BLOCK
    s = jnp.where(col < _COLS, s, jnp.float32(float("-inf")))

    m = jnp.max(s, axis=1, keepdims=True)
    a = jnp.min(jnp.where(s == m, col, jnp.int32(2**31 - 1)),
                axis=1, keepdims=True)

    @pl.when(b == 0)
    def _():
        max_ref[...] = m
        arg_ref[...] = a

    @pl.when(b > 0)
    def _():
        upd = m > max_ref[...]
        arg_ref[...] = jnp.where(upd, a, arg_ref[...])
        max_ref[...] = jnp.maximum(m, max_ref[...])

    @pl.when(b == _NBLK - 1)
    def _():
        out_ref[...] = arg_ref[...]


@jax.jit
def kernel(outputs):
    actions = pl.pallas_call(
        _sample_kernel,
        grid=(_NBLK,),
        in_specs=[
            pl.BlockSpec((_ROWS, _BLOCK), lambda b: (0, b)),
            pl.BlockSpec((1, _ROWS, _BLOCK), lambda b: (b, 0, 0)),
        ],
        out_specs=pl.BlockSpec((_ROWS, 1), lambda b: (0, 0)),
        out_shape=jax.ShapeDtypeStruct((_ROWS, 1), jnp.int32),
        scratch_shapes=[
            pltpu.VMEM((_ROWS, 1), jnp.float32),
            pltpu.VMEM((_ROWS, 1), jnp.int32),
        ],
    )(outputs, _BITS)
    return actions


# speedup vs baseline: 1.1753x; 1.0372x over previous
import numpy as np

import jax
import jax.numpy as jnp
from jax.experimental import pallas as pl
from jax.experimental.pallas import tpu as pltpu

_TEMP = 0.7
_ROWS = 128
_COLS = 100000
_BLOCK = 8192
_NBLK = (_COLS + _BLOCK - 1) // _BLOCK
_PAD_COLS = _NBLK * _BLOCK
_TINY = float(jnp.finfo(jnp.float32).tiny)


def _threefry_bits_table():
    ks1 = np.uint32(42)
    ks2 = np.uint32(0x1BD11BDA) ^ ks1
    rot = ((13, 15, 26, 6), (17, 29, 16, 24))
    inj = ((ks1, ks2), (ks2, np.uint32(0)), (np.uint32(0), ks1),
           (ks1, ks2), (ks2, np.uint32(0)))
    with np.errstate(over="ignore"):
        ctr = np.arange(_ROWS * _COLS, dtype=np.uint32)
        x0 = np.zeros_like(ctr)
        x1 = ctr + ks1
        for i in range(5):
            for r in rot[i % 2]:
                x0 += x1
                x1 = (x1 << np.uint32(r)) | (x1 >> np.uint32(32 - r))
                x1 ^= x0
            x0 += inj[i][0]
            x1 += inj[i][1] + np.uint32(i + 1)
        bits = (x0 ^ x1).reshape(_ROWS, _COLS)
    if _PAD_COLS != _COLS:
        bits = np.pad(bits, ((0, 0), (0, _PAD_COLS - _COLS)))
    return np.ascontiguousarray(
        bits.reshape(_ROWS, _NBLK, _BLOCK).transpose(1, 0, 2))


_BITS = _threefry_bits_table()


def _sample_kernel(x_ref, bits_ref, out_ref, max_ref, arg_ref):
    b = pl.program_id(0)

    bits = bits_ref[0]
    fb = (bits >> jnp.uint32(9)) | jnp.uint32(0x3F800000)
    u = jax.lax.bitcast_convert_type(fb, jnp.float32) - jnp.float32(1.0)
    u = jnp.maximum(jnp.float32(_TINY), u + jnp.float32(_TINY))
    g = -jnp.log(-jnp.log(u))

    s = x_ref[...] * jnp.float32(_TEMP) + g

    col = jax.lax.broadcasted_iota(jnp.int32, (_ROWS, _BLOCK), 1) + b * _BLOCK
    s = jnp.where(col < _COLS, s, jnp.float32(float("-inf")))

    m = jnp.max(s, axis=1, keepdims=True)
    a = jnp.min(jnp.where(s == m, col, jnp.int32(2**31 - 1)),
                axis=1, keepdims=True)

    @pl.when(b == 0)
    def _():
        max_ref[...] = m
        arg_ref[...] = a

    @pl.when(b > 0)
    def _():
        upd = m > max_ref[...]
        arg_ref[...] = jnp.where(upd, a, arg_ref[...])
        max_ref[...] = jnp.maximum(m, max_ref[...])

    @pl.when(b == _NBLK - 1)
    def _():
        out_ref[...] = arg_ref[...]


@jax.jit
def kernel(outputs):
    actions = pl.pallas_call(
        _sample_kernel,
        grid=(_NBLK,),
        in_specs=[
            pl.BlockSpec((_ROWS, _BLOCK), lambda b: (0, b)),
            pl.BlockSpec((1, _ROWS, _BLOCK), lambda b: (b, 0, 0)),
        ],
        out_specs=pl.BlockSpec((_ROWS, 1), lambda b: (0, 0)),
        out_shape=jax.ShapeDtypeStruct((_ROWS, 1), jnp.int32),
        scratch_shapes=[
            pltpu.VMEM((_ROWS, 1), jnp.float32),
            pltpu.VMEM((_ROWS, 1), jnp.int32),
        ],
    )(outputs, _BITS)
    return actions


# speedup vs baseline: 1.1908x; 1.0132x over previous
import numpy as np

import jax
import jax.numpy as jnp
from jax.experimental import pallas as pl
from jax.experimental.pallas import tpu as pltpu

_TEMP = 0.7
_ROWS = 128
_COLS = 100000
_BLOCK = 12800
_NBLK = (_COLS + _BLOCK - 1) // _BLOCK
_PAD_COLS = _NBLK * _BLOCK
_TINY = float(jnp.finfo(jnp.float32).tiny)


def _threefry_bits_table():
    ks1 = np.uint32(42)
    ks2 = np.uint32(0x1BD11BDA) ^ ks1
    rot = ((13, 15, 26, 6), (17, 29, 16, 24))
    inj = ((ks1, ks2), (ks2, np.uint32(0)), (np.uint32(0), ks1),
           (ks1, ks2), (ks2, np.uint32(0)))
    with np.errstate(over="ignore"):
        ctr = np.arange(_ROWS * _COLS, dtype=np.uint32)
        x0 = np.zeros_like(ctr)
        x1 = ctr + ks1
        for i in range(5):
            for r in rot[i % 2]:
                x0 += x1
                x1 = (x1 << np.uint32(r)) | (x1 >> np.uint32(32 - r))
                x1 ^= x0
            x0 += inj[i][0]
            x1 += inj[i][1] + np.uint32(i + 1)
        bits = (x0 ^ x1).reshape(_ROWS, _COLS)
    if _PAD_COLS != _COLS:
        bits = np.pad(bits, ((0, 0), (0, _PAD_COLS - _COLS)))
    return np.ascontiguousarray(
        bits.reshape(_ROWS, _NBLK, _BLOCK).transpose(1, 0, 2))


_BITS = _threefry_bits_table()


def _sample_kernel(x_ref, bits_ref, out_ref, max_ref, arg_ref):
    b = pl.program_id(0)

    bits = bits_ref[0]
    fb = (bits >> jnp.uint32(9)) | jnp.uint32(0x3F800000)
    u = jax.lax.bitcast_convert_type(fb, jnp.float32) - jnp.float32(1.0)
    u = jnp.maximum(jnp.float32(_TINY), u + jnp.float32(_TINY))
    g = -jnp.log(-jnp.log(u))

    s = x_ref[...] * jnp.float32(_TEMP) + g

    col = jax.lax.broadcasted_iota(jnp.int32, (_ROWS, _BLOCK), 1) + b * _BLOCK
    s = jnp.where(col < _COLS, s, jnp.float32(float("-inf")))

    m = jnp.max(s, axis=1, keepdims=True)
    a = jnp.min(jnp.where(s == m, col, jnp.int32(2**31 - 1)),
                axis=1, keepdims=True)

    @pl.when(b == 0)
    def _():
        max_ref[...] = m
        arg_ref[...] = a

    @pl.when(b > 0)
    def _():
        upd = m > max_ref[...]
        arg_ref[...] = jnp.where(upd, a, arg_ref[...])
        max_ref[...] = jnp.maximum(m, max_ref[...])

    @pl.when(b == _NBLK - 1)
    def _():
        out_ref[...] = arg_ref[...]


@jax.jit
def kernel(outputs):
    actions = pl.pallas_call(
        _sample_kernel,
        grid=(_NBLK,),
        in_specs=[
            pl.BlockSpec((_ROWS, _BLOCK), lambda b: (0, b)),
            pl.BlockSpec((1, _ROWS, _BLOCK), lambda b: (b, 0, 0)),
        ],
        out_specs=pl.BlockSpec((_ROWS, 1), lambda b: (0, 0)),
        out_shape=jax.ShapeDtypeStruct((_ROWS, 1), jnp.int32),
        scratch_shapes=[
            pltpu.VMEM((_ROWS, 1), jnp.float32),
            pltpu.VMEM((_ROWS, 1), jnp.int32),
        ],
    )(outputs, _BITS)
    return actions
